# Initial kernel scaffold; baseline (speedup 1.0000x reference)
import jax
import jax.numpy as jnp
from jax import lax
from jax.experimental import pallas as pl
from jax.experimental.pallas import tpu as pltpu
from jax.experimental.pallas import tpu_sc as plsc

N = 10000
E = 160000
T = 21
H = 16
NPAD = 10240
EPAD = 163840
EROWS = EPAD // 128
ROWS_PER_W = EROWS // 32
NTOK_CH = 80
STRIPE = NPAD // 16


def _tc_prep_body(ne, pw, pb, ee, w1, b1, w2, b2, srcr, tokr,
                  pt_o, ewt_o, gidx_o):
    pt_o[...] = jnp.maximum(
        jnp.dot(ne[...], pw[...], preferred_element_type=jnp.float32) + pb[...], 0.0)
    a = jnp.maximum(
        jnp.dot(ee[...], w1[...], preferred_element_type=jnp.float32) + b1[...], 0.0)
    ewt_o[...] = jnp.dot(a, w2[...], preferred_element_type=jnp.float32) + b2[...]
    gidx_o[...] = srcr[...] * T + tokr[...]


def _tc_pcat0_body(nf, wcat, pcat_o):
    pcat_o[...] = jnp.dot(nf[...], wcat[...], preferred_element_type=jnp.float32)


def _gru_core(agg2, h, cb, wir, wiz, win, whr, whz, whn,
              bir, biz, bin_, bhr, bhz, bhn):
    x = jnp.maximum(agg2[0] + agg2[1] + cb[...], 0.0)
    hh = h[...]

    def mm(v, w):
        return jnp.dot(v, w[...], preferred_element_type=jnp.float32)

    r = jax.nn.sigmoid(mm(x, wir) + bir[...] + mm(hh, whr) + bhr[...])
    z = jax.nn.sigmoid(mm(x, wiz) + biz[...] + mm(hh, whz) + bhz[...])
    n = jnp.tanh(mm(x, win) + bin_[...] + r * (mm(hh, whn) + bhn[...]))
    return (1.0 - z) * n + z * hh


def _tc_step_body(agg2, h, cb, wir, wiz, win, whr, whz, whn,
                  bir, biz, bin_, bhr, bhz, bhn, wcat, h_o, pcat_o):
    hn = _gru_core(agg2, h, cb, wir, wiz, win, whr, whz, whn,
                   bir, biz, bin_, bhr, bhz, bhn)
    h_o[...] = hn
    pcat_o[...] = jnp.dot(hn, wcat[...], preferred_element_type=jnp.float32)


def _tc_last_body(agg2, h, cb, wir, wiz, win, whr, whz, whn,
                  bir, biz, bin_, bhr, bhz, bhn, h_o):
    hn = _gru_core(agg2, h, cb, wir, wiz, win, whr, whz, whn,
                   bir, biz, bin_, bhr, bhz, bhn)
    h_o[...] = hn[:N]


def _sc_gather_nodes_body(pt_hbm, ntok_hbm, nf0_hbm, tokv, rows, sem):
    wid = lax.axis_index("s") * 2 + lax.axis_index("c")
    pltpu.sync_copy(ntok_hbm.at[pl.ds(wid * 4, 4)], tokv)
    for j in range(4):
        pltpu.async_copy(pt_hbm.at[tokv.at[j]], rows, sem).wait()
        pltpu.sync_copy(rows, nf0_hbm.at[pl.ds(wid * 320 + j * NTOK_CH, NTOK_CH)])


def _sc_msg_body(pcat_hbm, gidx_hbm, dstr_hbm, zero_hbm, agg2_hbm,
                 gv, dv, rows, agg_sh, sem):
    c = lax.axis_index("c")
    s = lax.axis_index("s")
    wid = s * 2 + c
    pltpu.sync_copy(zero_hbm, agg_sh.at[pl.ds(s * STRIPE, STRIPE)])
    pltpu.sync_copy(gidx_hbm.at[pl.ds(wid * ROWS_PER_W, ROWS_PER_W)], gv)
    pltpu.sync_copy(dstr_hbm.at[pl.ds(wid * ROWS_PER_W, ROWS_PER_W)], dv)
    plsc.subcore_barrier()

    def body(j, carry):
        pltpu.async_copy(pcat_hbm.at[gv.at[j]], rows, sem).wait()
        pltpu.sync_copy(rows, agg_sh.at[dv.at[j]], add=True)
        return carry

    lax.fori_loop(0, ROWS_PER_W, body, 0)
    plsc.subcore_barrier()
    pltpu.sync_copy(agg_sh.at[pl.ds(s * STRIPE, STRIPE)],
                    agg2_hbm.at[c, pl.ds(s * STRIPE, STRIPE)])


def kernel(node_tokens, edge_tokens, edge_index, node_emb, edge_emb,
           proj_W, proj_b, en_W1, en_b1, en_W2, en_b2, conv_bias,
           gru_Wih, gru_Whh, gru_bih, gru_bhh):
    src = edge_index[0].astype(jnp.int32)
    dst = edge_index[1].astype(jnp.int32)
    tok = edge_tokens.astype(jnp.int32)

    srcr = jnp.pad(src, (0, EPAD - E)).reshape(EROWS, 128)
    tokr = jnp.pad(tok, (0, EPAD - E)).reshape(EROWS, 128)
    dstr = jnp.pad(dst, (0, EPAD - E), constant_values=N).reshape(EROWS, 128)
    ntokr = jnp.pad(node_tokens.astype(jnp.int32), (0, NPAD - N)).reshape(128, NTOK_CH)
    zeros_stripe = jnp.zeros((STRIPE, H), jnp.float32)

    pb = proj_b.reshape(1, H)
    b1 = en_b1.reshape(1, 64)
    b2 = en_b2.reshape(1, H * H)
    cb = conv_bias.reshape(1, H)
    wir, wiz, win = (gru_Wih[0:H].T, gru_Wih[H:2 * H].T, gru_Wih[2 * H:3 * H].T)
    whr, whz, whn = (gru_Whh[0:H].T, gru_Whh[H:2 * H].T, gru_Whh[2 * H:3 * H].T)
    bir, biz, bin_ = (gru_bih[0:H].reshape(1, H), gru_bih[H:2 * H].reshape(1, H),
                      gru_bih[2 * H:3 * H].reshape(1, H))
    bhr, bhz, bhn = (gru_bhh[0:H].reshape(1, H), gru_bhh[H:2 * H].reshape(1, H),
                     gru_bhh[2 * H:3 * H].reshape(1, H))

    pt, ewt, gidx = pl.pallas_call(
        _tc_prep_body,
        out_shape=[
            jax.ShapeDtypeStruct((343, H), jnp.float32),
            jax.ShapeDtypeStruct((T, H * H), jnp.float32),
            jax.ShapeDtypeStruct((EROWS, 128), jnp.int32),
        ],
    )(node_emb, proj_W, pb, edge_emb, en_W1, b1, en_W2, b2, srcr, tokr)

    wcat = ewt.reshape(T, H, H).transpose(1, 0, 2).reshape(H, T * H)

    mesh = plsc.VectorSubcoreMesh(core_axis_name="c", subcore_axis_name="s")

    nf0 = pl.kernel(
        _sc_gather_nodes_body,
        out_type=jax.ShapeDtypeStruct((NPAD, H), jnp.float32),
        mesh=mesh,
        scratch_types=[
            pltpu.VMEM((4, NTOK_CH), jnp.int32),
            pltpu.VMEM((NTOK_CH, H), jnp.float32),
            pltpu.SemaphoreType.DMA,
        ],
    )(pt, ntokr)

    sc_msg = pl.kernel(
        _sc_msg_body,
        out_type=jax.ShapeDtypeStruct((2, NPAD, H), jnp.float32),
        mesh=mesh,
        scratch_types=[
            pltpu.VMEM((ROWS_PER_W, 128), jnp.int32),
            pltpu.VMEM((ROWS_PER_W, 128), jnp.int32),
            pltpu.VMEM((128, H), jnp.float32),
            pltpu.VMEM_SHARED((NPAD, H), jnp.float32),
            pltpu.SemaphoreType.DMA,
        ],
    )

    tc_step = pl.pallas_call(
        _tc_step_body,
        out_shape=[
            jax.ShapeDtypeStruct((NPAD, H), jnp.float32),
            jax.ShapeDtypeStruct((NPAD, T * H), jnp.float32),
        ],
    )
    tc_last = pl.pallas_call(
        _tc_last_body,
        out_shape=jax.ShapeDtypeStruct((N, H), jnp.float32),
    )

    pcat = pl.pallas_call(
        _tc_pcat0_body,
        out_shape=jax.ShapeDtypeStruct((NPAD, T * H), jnp.float32),
    )(nf0, wcat)

    h = nf0
    gru_args = (cb, wir, wiz, win, whr, whz, whn, bir, biz, bin_, bhr, bhz, bhn)
    for step in range(3):
        agg2 = sc_msg(pcat.reshape(NPAD * T, H), gidx, dstr, zeros_stripe)
        if step < 2:
            h, pcat = tc_step(agg2, h, *gru_args, wcat)
        else:
            return tc_last(agg2, h, *gru_args)


# speedup vs baseline: 10.4229x; 10.4229x over previous
import jax
import jax.numpy as jnp
from jax import lax
from jax.experimental import pallas as pl
from jax.experimental.pallas import tpu as pltpu
from jax.experimental.pallas import tpu_sc as plsc

N = 10000
E = 160000
T = 21
H = 16
NPAD = 10240
EPAD = 163840
EROWS = EPAD // 128
ROWS_PER_W = EROWS // 32
NTOK_CH = 80
STRIPE = NPAD // 16


def _tc_prep_body(ne, pw, pb, ee, w1, b1, w2, b2, srcr, tokr,
                  pt_o, ewt_o, gidx_o):
    pt_o[...] = jnp.maximum(
        jnp.dot(ne[...], pw[...], preferred_element_type=jnp.float32) + pb[...], 0.0)
    a = jnp.maximum(
        jnp.dot(ee[...], w1[...], preferred_element_type=jnp.float32) + b1[...], 0.0)
    ewt_o[...] = jnp.dot(a, w2[...], preferred_element_type=jnp.float32) + b2[...]
    gidx_o[...] = srcr[...] * T + tokr[...]


def _tc_pcat0_body(nf, wcat, pcat_o):
    pcat_o[...] = jnp.dot(nf[...], wcat[...], preferred_element_type=jnp.float32)


def _gru_core(agg2, h, cb, wir, wiz, win, whr, whz, whn,
              bir, biz, bin_, bhr, bhz, bhn):
    x = jnp.maximum(agg2[0] + agg2[1] + cb[...], 0.0)
    hh = h[...]

    def mm(v, w):
        return jnp.dot(v, w[...], preferred_element_type=jnp.float32)

    r = jax.nn.sigmoid(mm(x, wir) + bir[...] + mm(hh, whr) + bhr[...])
    z = jax.nn.sigmoid(mm(x, wiz) + biz[...] + mm(hh, whz) + bhz[...])
    n = jnp.tanh(mm(x, win) + bin_[...] + r * (mm(hh, whn) + bhn[...]))
    return (1.0 - z) * n + z * hh


def _tc_step_body(agg2, h, cb, wir, wiz, win, whr, whz, whn,
                  bir, biz, bin_, bhr, bhz, bhn, wcat, h_o, pcat_o):
    hn = _gru_core(agg2, h, cb, wir, wiz, win, whr, whz, whn,
                   bir, biz, bin_, bhr, bhz, bhn)
    h_o[...] = hn
    pcat_o[...] = jnp.dot(hn, wcat[...], preferred_element_type=jnp.float32)


def _tc_last_body(agg2, h, cb, wir, wiz, win, whr, whz, whn,
                  bir, biz, bin_, bhr, bhz, bhn, h_o):
    hn = _gru_core(agg2, h, cb, wir, wiz, win, whr, whz, whn,
                   bir, biz, bin_, bhr, bhz, bhn)
    h_o[...] = hn[:N]


def _sc_gather_nodes_body(pt_hbm, ntok_hbm, nf0_hbm, tokv, rows, sem):
    wid = lax.axis_index("s") * 2 + lax.axis_index("c")
    pltpu.sync_copy(ntok_hbm.at[pl.ds(wid * 4, 4)], tokv)
    for j in range(4):
        pltpu.async_copy(pt_hbm.at[tokv.at[j]], rows, sem).wait()
        pltpu.sync_copy(rows, nf0_hbm.at[pl.ds(wid * 320 + j * NTOK_CH, NTOK_CH)])


def _sc_msg_body(pcat_hbm, gidx_hbm, dstr_hbm, zero_hbm, agg2_hbm,
                 gv, dv, rows, agg_sh, sem):
    c = lax.axis_index("c")
    s = lax.axis_index("s")
    wid = s * 2 + c
    pltpu.sync_copy(zero_hbm, agg_sh.at[pl.ds(s * STRIPE, STRIPE)])
    pltpu.sync_copy(gidx_hbm.at[pl.ds(wid * ROWS_PER_W, ROWS_PER_W)], gv)
    pltpu.sync_copy(dstr_hbm.at[pl.ds(wid * ROWS_PER_W, ROWS_PER_W)], dv)
    plsc.subcore_barrier()

    def body(j, carry):
        pltpu.async_copy(pcat_hbm.at[gv.at[j]], rows, sem).wait()
        pltpu.sync_copy(rows, agg_sh.at[dv.at[j]], add=True)
        return carry

    lax.fori_loop(0, ROWS_PER_W, body, 0)
    plsc.subcore_barrier()
    pltpu.sync_copy(agg_sh.at[pl.ds(s * STRIPE, STRIPE)],
                    agg2_hbm.at[c, pl.ds(s * STRIPE, STRIPE)])


def kernel(node_tokens, edge_tokens, edge_index, node_emb, edge_emb,
           proj_W, proj_b, en_W1, en_b1, en_W2, en_b2, conv_bias,
           gru_Wih, gru_Whh, gru_bih, gru_bhh):
    src = edge_index[0].astype(jnp.int32)
    dst = edge_index[1].astype(jnp.int32)
    tok = edge_tokens.astype(jnp.int32)

    srcr = jnp.pad(src, (0, EPAD - E)).reshape(EROWS, 128)
    tokr = jnp.pad(tok, (0, EPAD - E)).reshape(EROWS, 128)
    dstr = jnp.pad(dst, (0, EPAD - E), constant_values=N).reshape(EROWS, 128)
    ntokr = jnp.pad(node_tokens.astype(jnp.int32), (0, NPAD - N)).reshape(128, NTOK_CH)
    zeros_stripe = jnp.zeros((STRIPE, H), jnp.float32)

    pb = proj_b.reshape(1, H)
    b1 = en_b1.reshape(1, 64)
    b2 = en_b2.reshape(1, H * H)
    cb = conv_bias.reshape(1, H)
    wir, wiz, win = (gru_Wih[0:H].T, gru_Wih[H:2 * H].T, gru_Wih[2 * H:3 * H].T)
    whr, whz, whn = (gru_Whh[0:H].T, gru_Whh[H:2 * H].T, gru_Whh[2 * H:3 * H].T)
    bir, biz, bin_ = (gru_bih[0:H].reshape(1, H), gru_bih[H:2 * H].reshape(1, H),
                      gru_bih[2 * H:3 * H].reshape(1, H))
    bhr, bhz, bhn = (gru_bhh[0:H].reshape(1, H), gru_bhh[H:2 * H].reshape(1, H),
                     gru_bhh[2 * H:3 * H].reshape(1, H))

    pt, ewt, gidx = pl.pallas_call(
        _tc_prep_body,
        out_shape=[
            jax.ShapeDtypeStruct((343, H), jnp.float32),
            jax.ShapeDtypeStruct((T, H * H), jnp.float32),
            jax.ShapeDtypeStruct((EROWS, 128), jnp.int32),
        ],
    )(node_emb, proj_W, pb, edge_emb, en_W1, b1, en_W2, b2, srcr, tokr)

    wcat = ewt.reshape(T, H, H).transpose(1, 0, 2).reshape(H, T * H)

    mesh = plsc.VectorSubcoreMesh(core_axis_name="c", subcore_axis_name="s")
    sc_params = pltpu.CompilerParams(use_tc_tiling_on_sc=False)

    nf0 = pl.kernel(
        _sc_gather_nodes_body,
        out_type=jax.ShapeDtypeStruct((NPAD, H), jnp.float32),
        mesh=mesh,
        compiler_params=sc_params,
        scratch_types=[
            pltpu.VMEM((4, NTOK_CH), jnp.int32),
            pltpu.VMEM((NTOK_CH, H), jnp.float32),
            pltpu.SemaphoreType.DMA,
        ],
    )(pt, ntokr)

    sc_msg = pl.kernel(
        _sc_msg_body,
        out_type=jax.ShapeDtypeStruct((2, NPAD, H), jnp.float32),
        mesh=mesh,
        compiler_params=sc_params,
        scratch_types=[
            pltpu.VMEM((ROWS_PER_W, 128), jnp.int32),
            pltpu.VMEM((ROWS_PER_W, 128), jnp.int32),
            pltpu.VMEM((128, H), jnp.float32),
            pltpu.VMEM_SHARED((NPAD, H), jnp.float32),
            pltpu.SemaphoreType.DMA,
        ],
    )

    tc_step = pl.pallas_call(
        _tc_step_body,
        out_shape=[
            jax.ShapeDtypeStruct((NPAD, H), jnp.float32),
            jax.ShapeDtypeStruct((NPAD, T * H), jnp.float32),
        ],
    )
    tc_last = pl.pallas_call(
        _tc_last_body,
        out_shape=jax.ShapeDtypeStruct((N, H), jnp.float32),
    )

    pcat = pl.pallas_call(
        _tc_pcat0_body,
        out_shape=jax.ShapeDtypeStruct((NPAD, T * H), jnp.float32),
    )(nf0, wcat)

    h = nf0
    gru_args = (cb, wir, wiz, win, whr, whz, whn, bir, biz, bin_, bhr, bhz, bhn)
    for step in range(3):
        agg2 = sc_msg(pcat.reshape(NPAD * T, H), gidx, dstr, zeros_stripe)
        if step < 2:
            h, pcat = tc_step(agg2, h, *gru_args, wcat)
        else:
            return tc_last(agg2, h, *gru_args)


# speedup vs baseline: 12.4088x; 1.1905x over previous
import jax
import jax.numpy as jnp
from jax import lax
from jax.experimental import pallas as pl
from jax.experimental.pallas import tpu as pltpu
from jax.experimental.pallas import tpu_sc as plsc

N = 10000
E = 160000
T = 21
H = 16
NPAD = 10240
EPAD = 163840
EROWS = EPAD // 128
ROWS_PER_W = EROWS // 32
NTOK_CH = 80
STRIPE = NPAD // 16


def _tc_prep_body(ne, pw, pb, ee, w1, b1, w2, b2, srcr, tokr,
                  pt_o, ewt_o, gidx_o):
    pt_o[...] = jnp.maximum(
        jnp.dot(ne[...], pw[...], preferred_element_type=jnp.float32) + pb[...], 0.0)
    a = jnp.maximum(
        jnp.dot(ee[...], w1[...], preferred_element_type=jnp.float32) + b1[...], 0.0)
    ewt_o[...] = jnp.dot(a, w2[...], preferred_element_type=jnp.float32) + b2[...]
    gidx_o[...] = srcr[...] * T + tokr[...]


def _tc_pcat0_body(nf, wcat, pcat_o):
    pcat_o[...] = jnp.dot(nf[...], wcat[...], preferred_element_type=jnp.float32)


def _gru_core(agg2, h, cb, wir, wiz, win, whr, whz, whn,
              bir, biz, bin_, bhr, bhz, bhn):
    x = jnp.maximum(agg2[0] + agg2[1] + cb[...], 0.0)
    hh = h[...]

    def mm(v, w):
        return jnp.dot(v, w[...], preferred_element_type=jnp.float32)

    r = jax.nn.sigmoid(mm(x, wir) + bir[...] + mm(hh, whr) + bhr[...])
    z = jax.nn.sigmoid(mm(x, wiz) + biz[...] + mm(hh, whz) + bhz[...])
    n = jnp.tanh(mm(x, win) + bin_[...] + r * (mm(hh, whn) + bhn[...]))
    return (1.0 - z) * n + z * hh


def _tc_step_body(agg2, h, cb, wir, wiz, win, whr, whz, whn,
                  bir, biz, bin_, bhr, bhz, bhn, wcat, h_o, pcat_o):
    hn = _gru_core(agg2, h, cb, wir, wiz, win, whr, whz, whn,
                   bir, biz, bin_, bhr, bhz, bhn)
    h_o[...] = hn
    pcat_o[...] = jnp.dot(hn, wcat[...], preferred_element_type=jnp.float32)


def _tc_last_body(agg2, h, cb, wir, wiz, win, whr, whz, whn,
                  bir, biz, bin_, bhr, bhz, bhn, h_o):
    hn = _gru_core(agg2, h, cb, wir, wiz, win, whr, whz, whn,
                   bir, biz, bin_, bhr, bhz, bhn)
    h_o[...] = hn[:N]


def _sc_gather_nodes_body(pt_hbm, ntok_hbm, nf0_hbm, tokv, rows, sem):
    wid = lax.axis_index("s") * 2 + lax.axis_index("c")
    pltpu.sync_copy(ntok_hbm.at[pl.ds(wid * 4, 4)], tokv)
    for j in range(4):
        pltpu.async_copy(pt_hbm.at[tokv.at[j]], rows, sem).wait()
        pltpu.sync_copy(rows, nf0_hbm.at[pl.ds(wid * 320 + j * NTOK_CH, NTOK_CH)])


RING = 4


def _sc_msg_body(pcat_hbm, gidx_hbm, dstr_hbm, zero_hbm, agg2_hbm,
                 gv, dv, rows, agg_sh, sems):
    c = lax.axis_index("c")
    s = lax.axis_index("s")
    wid = s * 2 + c
    pltpu.sync_copy(zero_hbm, agg_sh.at[pl.ds(s * STRIPE, STRIPE)])
    pltpu.sync_copy(gidx_hbm.at[pl.ds(wid * ROWS_PER_W, ROWS_PER_W)], gv)
    pltpu.sync_copy(dstr_hbm.at[pl.ds(wid * ROWS_PER_W, ROWS_PER_W)], dv)
    plsc.subcore_barrier()

    for b in range(RING):
        pltpu.async_copy(pcat_hbm.at[gv.at[b]], rows.at[b], sems.at[b])

    @pl.loop(0, ROWS_PER_W, step=RING)
    def _chunk(g):
        for b in range(RING):
            j = g + b
            pltpu.make_async_copy(pcat_hbm.at[gv.at[0]], rows.at[b],
                                  sems.at[b]).wait()
            pltpu.sync_copy(rows.at[b], agg_sh.at[dv.at[j]], add=True)

            @pl.when(j + RING < ROWS_PER_W)
            def _refill():
                pltpu.async_copy(pcat_hbm.at[gv.at[j + RING]], rows.at[b],
                                 sems.at[b])

    plsc.subcore_barrier()
    pltpu.sync_copy(agg_sh.at[pl.ds(s * STRIPE, STRIPE)],
                    agg2_hbm.at[c, pl.ds(s * STRIPE, STRIPE)])


def kernel(node_tokens, edge_tokens, edge_index, node_emb, edge_emb,
           proj_W, proj_b, en_W1, en_b1, en_W2, en_b2, conv_bias,
           gru_Wih, gru_Whh, gru_bih, gru_bhh):
    src = edge_index[0].astype(jnp.int32)
    dst = edge_index[1].astype(jnp.int32)
    tok = edge_tokens.astype(jnp.int32)

    srcr = jnp.pad(src, (0, EPAD - E)).reshape(EROWS, 128)
    tokr = jnp.pad(tok, (0, EPAD - E)).reshape(EROWS, 128)
    dstr = jnp.pad(dst, (0, EPAD - E), constant_values=N).reshape(EROWS, 128)
    ntokr = jnp.pad(node_tokens.astype(jnp.int32), (0, NPAD - N)).reshape(128, NTOK_CH)
    zeros_stripe = jnp.zeros((STRIPE, H), jnp.float32)

    pb = proj_b.reshape(1, H)
    b1 = en_b1.reshape(1, 64)
    b2 = en_b2.reshape(1, H * H)
    cb = conv_bias.reshape(1, H)
    wir, wiz, win = (gru_Wih[0:H].T, gru_Wih[H:2 * H].T, gru_Wih[2 * H:3 * H].T)
    whr, whz, whn = (gru_Whh[0:H].T, gru_Whh[H:2 * H].T, gru_Whh[2 * H:3 * H].T)
    bir, biz, bin_ = (gru_bih[0:H].reshape(1, H), gru_bih[H:2 * H].reshape(1, H),
                      gru_bih[2 * H:3 * H].reshape(1, H))
    bhr, bhz, bhn = (gru_bhh[0:H].reshape(1, H), gru_bhh[H:2 * H].reshape(1, H),
                     gru_bhh[2 * H:3 * H].reshape(1, H))

    pt, ewt, gidx = pl.pallas_call(
        _tc_prep_body,
        out_shape=[
            jax.ShapeDtypeStruct((343, H), jnp.float32),
            jax.ShapeDtypeStruct((T, H * H), jnp.float32),
            jax.ShapeDtypeStruct((EROWS, 128), jnp.int32),
        ],
    )(node_emb, proj_W, pb, edge_emb, en_W1, b1, en_W2, b2, srcr, tokr)

    wcat = ewt.reshape(T, H, H).transpose(1, 0, 2).reshape(H, T * H)

    mesh = plsc.VectorSubcoreMesh(core_axis_name="c", subcore_axis_name="s")
    sc_params = pltpu.CompilerParams(use_tc_tiling_on_sc=False)

    nf0 = pl.kernel(
        _sc_gather_nodes_body,
        out_type=jax.ShapeDtypeStruct((NPAD, H), jnp.float32),
        mesh=mesh,
        compiler_params=sc_params,
        scratch_types=[
            pltpu.VMEM((4, NTOK_CH), jnp.int32),
            pltpu.VMEM((NTOK_CH, H), jnp.float32),
            pltpu.SemaphoreType.DMA,
        ],
    )(pt, ntokr)

    sc_msg = pl.kernel(
        _sc_msg_body,
        out_type=jax.ShapeDtypeStruct((2, NPAD, H), jnp.float32),
        mesh=mesh,
        compiler_params=sc_params,
        scratch_types=[
            pltpu.VMEM((ROWS_PER_W, 128), jnp.int32),
            pltpu.VMEM((ROWS_PER_W, 128), jnp.int32),
            pltpu.VMEM((RING, 128, H), jnp.float32),
            pltpu.VMEM_SHARED((NPAD, H), jnp.float32),
            pltpu.SemaphoreType.DMA((RING,)),
        ],
    )

    tc_step = pl.pallas_call(
        _tc_step_body,
        out_shape=[
            jax.ShapeDtypeStruct((NPAD, H), jnp.float32),
            jax.ShapeDtypeStruct((NPAD, T * H), jnp.float32),
        ],
    )
    tc_last = pl.pallas_call(
        _tc_last_body,
        out_shape=jax.ShapeDtypeStruct((N, H), jnp.float32),
    )

    pcat = pl.pallas_call(
        _tc_pcat0_body,
        out_shape=jax.ShapeDtypeStruct((NPAD, T * H), jnp.float32),
    )(nf0, wcat)

    h = nf0
    gru_args = (cb, wir, wiz, win, whr, whz, whn, bir, biz, bin_, bhr, bhz, bhn)
    for step in range(3):
        agg2 = sc_msg(pcat.reshape(NPAD * T, H), gidx, dstr, zeros_stripe)
        if step < 2:
            h, pcat = tc_step(agg2, h, *gru_args, wcat)
        else:
            return tc_last(agg2, h, *gru_args)
